# baseline (speedup 1.0000x reference)
import functools

import jax
import jax.numpy as jnp
from jax import lax
from jax.experimental import pallas as pl
from jax.experimental.pallas import tpu as pltpu
from jax.experimental.pallas import tpu_sc as plsc

VOCAB = 32128
D_MODEL = 1024
BATCH = 4
SEQ = 2048

NC, NS, L = 2, 16, 16
NW = NC * NS
TOTAL = BATCH * SEQ
B_PER_W = TOTAL // NW
W_PER_B = SEQ // B_PER_W
CHUNK = 16
NCH = B_PER_W // CHUNK
NBUF = 4

_F32_MIN = float(jnp.finfo(jnp.float32).min)


def _sc_body(ids_hbm, mask_hbm, table_hbm, out_hbm, ext_hbm,
             idx_v, mask_v, ext_v, bufs, gsems, wsems):
    wid = lax.axis_index("s") * NC + lax.axis_index("c")
    base = wid * B_PER_W
    row = wid // W_PER_B
    col = (wid % W_PER_B) * B_PER_W

    pltpu.sync_copy(ids_hbm.at[row, pl.ds(col, B_PER_W)], idx_v)

    def gather(j):
        return pltpu.async_copy(
            table_hbm.at[idx_v.at[pl.ds(j * CHUNK, CHUNK)]],
            bufs[j % NBUF], gsems[j % NBUF])

    gcopies = {j: gather(j) for j in range(NBUF - 1)}

    pltpu.sync_copy(mask_hbm.at[row, pl.ds(col, B_PER_W)], mask_v)
    for m in range(B_PER_W // L):
        v = mask_v[pl.ds(m * L, L)].astype(jnp.float32)
        ext_v[pl.ds(m * L, L)] = (1.0 - v) * _F32_MIN
    pltpu.sync_copy(ext_v, ext_hbm.at[pl.ds(base, B_PER_W)])

    wcopies = {}
    for j in range(NCH):
        if j >= 1:
            wcopies[j - 1].wait()
        if j + NBUF - 1 < NCH:
            gcopies[j + NBUF - 1] = gather(j + NBUF - 1)
        gcopies[j].wait()
        wcopies[j] = pltpu.async_copy(
            bufs[j % NBUF], out_hbm.at[pl.ds(base + j * CHUNK, CHUNK)],
            wsems[j % NBUF])
    wcopies[NCH - 1].wait()


@jax.jit
def _sc_embed(ids, mask, table):
    mesh = plsc.VectorSubcoreMesh(core_axis_name="c", subcore_axis_name="s")
    fn = pl.kernel(
        _sc_body,
        out_type=[
            jax.ShapeDtypeStruct((TOTAL, D_MODEL), jnp.float32),
            jax.ShapeDtypeStruct((TOTAL,), jnp.float32),
        ],
        mesh=mesh,
        scratch_types=[
            pltpu.VMEM((B_PER_W,), jnp.int32),
            pltpu.VMEM((B_PER_W,), jnp.int32),
            pltpu.VMEM((B_PER_W,), jnp.float32),
            [pltpu.VMEM((CHUNK, D_MODEL), jnp.float32) for _ in range(NBUF)],
            [pltpu.SemaphoreType.DMA for _ in range(NBUF)],
            [pltpu.SemaphoreType.DMA for _ in range(NBUF)],
        ],
        name="t5_embed_gather_sc",
    )
    return fn(ids, mask, table)


def kernel(encoder_input_ids, encoder_attention_mask, embed_table):
    ids = encoder_input_ids.astype(jnp.int32)
    mask = encoder_attention_mask.astype(jnp.int32)
    hidden_flat, ext_flat = _sc_embed(ids, mask, embed_table)
    hidden = hidden_flat.reshape(BATCH, SEQ, D_MODEL)
    ext = ext_flat.reshape(BATCH, 1, 1, SEQ)
    return (encoder_attention_mask, ext, hidden)


# speedup vs baseline: 1.0131x; 1.0131x over previous
import functools

import jax
import jax.numpy as jnp
from jax import lax
from jax.experimental import pallas as pl
from jax.experimental.pallas import tpu as pltpu
from jax.experimental.pallas import tpu_sc as plsc

VOCAB = 32128
D_MODEL = 1024
BATCH = 4
SEQ = 2048

NC, NS, L = 2, 16, 16
NW = NC * NS
TOTAL = BATCH * SEQ
B_PER_W = TOTAL // NW
W_PER_B = SEQ // B_PER_W
CHUNK = 32
NCH = B_PER_W // CHUNK
NBUF = 3

_F32_MIN = float(jnp.finfo(jnp.float32).min)


def _sc_body(ids_hbm, mask_hbm, table_hbm, out_hbm, ext_hbm,
             idx_v, mask_v, ext_v, bufs, gsems, wsems, esem):
    wid = lax.axis_index("s") * NC + lax.axis_index("c")
    base = wid * B_PER_W
    row = wid // W_PER_B
    col = (wid % W_PER_B) * B_PER_W

    pltpu.sync_copy(ids_hbm.at[row, pl.ds(col, B_PER_W)], idx_v)

    def gather(j):
        return pltpu.async_copy(
            table_hbm.at[idx_v.at[pl.ds(j * CHUNK, CHUNK)]],
            bufs[j % NBUF], gsems[j % NBUF])

    gcopies = {j: gather(j) for j in range(NBUF - 1)}

    pltpu.sync_copy(mask_hbm.at[row, pl.ds(col, B_PER_W)], mask_v)
    for m in range(B_PER_W // L):
        v = mask_v[pl.ds(m * L, L)].astype(jnp.float32)
        ext_v[pl.ds(m * L, L)] = (1.0 - v) * _F32_MIN
    ecopy = pltpu.async_copy(ext_v, ext_hbm.at[pl.ds(base, B_PER_W)], esem)

    wcopies = {}
    for j in range(NCH):
        if j >= 1:
            wcopies[j - 1].wait()
        if j + NBUF - 1 < NCH:
            gcopies[j + NBUF - 1] = gather(j + NBUF - 1)
        gcopies[j].wait()
        wcopies[j] = pltpu.async_copy(
            bufs[j % NBUF], out_hbm.at[pl.ds(base + j * CHUNK, CHUNK)],
            wsems[j % NBUF])
    wcopies[NCH - 1].wait()
    ecopy.wait()


@jax.jit
def _sc_embed(ids, mask, table):
    mesh = plsc.VectorSubcoreMesh(core_axis_name="c", subcore_axis_name="s")
    fn = pl.kernel(
        _sc_body,
        out_type=[
            jax.ShapeDtypeStruct((TOTAL, D_MODEL), jnp.float32),
            jax.ShapeDtypeStruct((TOTAL,), jnp.float32),
        ],
        mesh=mesh,
        scratch_types=[
            pltpu.VMEM((B_PER_W,), jnp.int32),
            pltpu.VMEM((B_PER_W,), jnp.int32),
            pltpu.VMEM((B_PER_W,), jnp.float32),
            [pltpu.VMEM((CHUNK, D_MODEL), jnp.float32) for _ in range(NBUF)],
            [pltpu.SemaphoreType.DMA for _ in range(NBUF)],
            [pltpu.SemaphoreType.DMA for _ in range(NBUF)],
            pltpu.SemaphoreType.DMA,
        ],
        name="t5_embed_gather_sc",
    )
    return fn(ids, mask, table)


def kernel(encoder_input_ids, encoder_attention_mask, embed_table):
    ids = encoder_input_ids.astype(jnp.int32)
    mask = encoder_attention_mask.astype(jnp.int32)
    hidden_flat, ext_flat = _sc_embed(ids, mask, embed_table)
    hidden = hidden_flat.reshape(BATCH, SEQ, D_MODEL)
    ext = ext_flat.reshape(BATCH, 1, 1, SEQ)
    return (encoder_attention_mask, ext, hidden)
